# baseline (speedup 1.0000x reference)
import functools

import jax
import jax.numpy as jnp
from jax import lax
from jax.experimental import pallas as pl
from jax.experimental.pallas import tpu as pltpu
from jax.experimental.pallas import tpu_sc as plsc

BUFFER_SIZE = 10000
N_CLASSES = 100
BATCH = 4096
ROW = 3 * 32 * 32

R1 = BATCH * ROW
R2 = (BUFFER_SIZE - BATCH) * ROW
TOT = BUFFER_SIZE * ROW
TAIL = BUFFER_SIZE - BATCH

NDMA = 8
S1 = R1 // NDMA
S2 = R2 // NDMA


def _bx_body(x_h, bx_h, obx_h, sems):
    ds = []
    for k in range(NDMA):
        off = k * S1
        ds.append(pltpu.make_async_copy(
            x_h.at[pl.ds(off, S1)], obx_h.at[pl.ds(off, S1)], sems.at[k]))
    for k in range(NDMA):
        off = R1 + k * S2
        ds.append(pltpu.make_async_copy(
            bx_h.at[pl.ds(off, S2)], obx_h.at[pl.ds(off, S2)],
            sems.at[NDMA + k]))
    for d in ds:
        d.start()
    for d in ds:
        d.wait()


def _sc_body(y_h, t_h, by_h, bt_h, oby_h, obt_h, stage, tvec, sem0, sem1):
    cid = lax.axis_index("c")
    sid = lax.axis_index("s")
    wid = sid * 2 + cid

    @pl.when(wid == 0)
    def _():
        d0 = pltpu.async_copy(y_h, stage.at[pl.ds(0, BATCH)], sem0)
        d1 = pltpu.async_copy(by_h.at[pl.ds(BATCH, TAIL)],
                              stage.at[pl.ds(BATCH, TAIL)], sem1)
        d0.wait()
        pltpu.async_copy(stage.at[pl.ds(0, BATCH)],
                         oby_h.at[pl.ds(0, BATCH)], sem0).wait()
        d1.wait()
        pltpu.async_copy(stage.at[pl.ds(BATCH, TAIL)],
                         oby_h.at[pl.ds(BATCH, TAIL)], sem1).wait()

    @pl.when(wid == 1)
    def _():
        pltpu.sync_copy(t_h, tvec)
        d1 = pltpu.async_copy(bt_h.at[pl.ds(BATCH, TAIL)],
                              stage.at[pl.ds(BATCH, TAIL)], sem1)
        tv = tvec[...]
        for i in range(BATCH // 16):
            stage[pl.ds(i * 16, 16)] = tv
        pltpu.async_copy(stage.at[pl.ds(0, BATCH)],
                         obt_h.at[pl.ds(0, BATCH)], sem0).wait()
        d1.wait()
        pltpu.async_copy(stage.at[pl.ds(BATCH, TAIL)],
                         obt_h.at[pl.ds(BATCH, TAIL)], sem1).wait()


@jax.jit
def _er_update(x, y, t16, bx, by, bt):
    xf = x.reshape(R1)
    bxf = bx.reshape(TOT)

    obx = pl.pallas_call(
        _bx_body,
        in_specs=[pl.BlockSpec(memory_space=pltpu.MemorySpace.HBM),
                  pl.BlockSpec(memory_space=pltpu.MemorySpace.HBM)],
        out_specs=pl.BlockSpec(memory_space=pltpu.MemorySpace.HBM),
        out_shape=jax.ShapeDtypeStruct((TOT,), jnp.float32),
        scratch_shapes=[pltpu.SemaphoreType.DMA((2 * NDMA,))],
    )(xf, bxf)

    mesh = plsc.VectorSubcoreMesh(core_axis_name="c", subcore_axis_name="s")
    oby, obt = pl.kernel(
        _sc_body,
        out_type=(
            jax.ShapeDtypeStruct((BUFFER_SIZE,), jnp.int32),
            jax.ShapeDtypeStruct((BUFFER_SIZE,), jnp.int32),
        ),
        mesh=mesh,
        scratch_types=[
            pltpu.VMEM((BUFFER_SIZE,), jnp.int32),
            pltpu.VMEM((16,), jnp.int32),
            pltpu.SemaphoreType.DMA,
            pltpu.SemaphoreType.DMA,
        ],
    )(y, t16, by, bt)
    return obx.reshape(bx.shape), oby, obt


def kernel(x, y, task_id, bx, by, bt):
    t16 = jnp.full((16,), task_id, dtype=jnp.int32)
    return _er_update(x, y, t16, bx, by, bt)


# speedup vs baseline: 4.2250x; 4.2250x over previous
import functools

import jax
import jax.numpy as jnp
from jax import lax
from jax.experimental import pallas as pl
from jax.experimental.pallas import tpu as pltpu
from jax.experimental.pallas import tpu_sc as plsc

BUFFER_SIZE = 10000
N_CLASSES = 100
BATCH = 4096
ROW = 3 * 32 * 32

R1 = BATCH * ROW
R2 = (BUFFER_SIZE - BATCH) * ROW
TOT = BUFFER_SIZE * ROW
TAIL = BUFFER_SIZE - BATCH

CHUNK = 1_048_576
NB = 6
DEPTH = 3


def _bx_body(x_h, bx_h, obx_h, *bufs_and_sems):
    bufs = bufs_and_sems[:NB]
    sin = bufs_and_sems[NB]
    sout = bufs_and_sems[NB + 1]

    chunks = []
    for off in range(0, R1, CHUNK):
        chunks.append((x_h, off, min(CHUNK, R1 - off)))
    for off in range(R1, TOT, CHUNK):
        chunks.append((bx_h, off, min(CHUNK, TOT - off)))
    n = len(chunks)

    in_d = [None] * n
    out_d = [None] * n

    def start_in(i):
        src, off, sz = chunks[i]
        b = i % NB
        in_d[i] = pltpu.make_async_copy(
            src.at[pl.ds(off, sz)], bufs[b].at[pl.ds(0, sz)], sin.at[b])
        in_d[i].start()

    for i in range(min(DEPTH, n)):
        start_in(i)
    for i in range(n):
        _, off, sz = chunks[i]
        b = i % NB
        in_d[i].wait()
        out_d[i] = pltpu.make_async_copy(
            bufs[b].at[pl.ds(0, sz)], obx_h.at[pl.ds(off, sz)], sout.at[b])
        out_d[i].start()
        j = i + DEPTH
        if j < n:
            if j >= NB:
                out_d[j - NB].wait()
            start_in(j)
    for i in range(max(0, n - NB), n):
        out_d[i].wait()


def _sc_body(y_h, t_h, by_h, bt_h, oby_h, obt_h, stage, tvec, sem0, sem1):
    cid = lax.axis_index("c")
    sid = lax.axis_index("s")
    wid = sid * 2 + cid

    @pl.when(wid == 0)
    def _():
        d0 = pltpu.async_copy(y_h, stage.at[pl.ds(0, BATCH)], sem0)
        d1 = pltpu.async_copy(by_h.at[pl.ds(BATCH, TAIL)],
                              stage.at[pl.ds(BATCH, TAIL)], sem1)
        d0.wait()
        pltpu.async_copy(stage.at[pl.ds(0, BATCH)],
                         oby_h.at[pl.ds(0, BATCH)], sem0).wait()
        d1.wait()
        pltpu.async_copy(stage.at[pl.ds(BATCH, TAIL)],
                         oby_h.at[pl.ds(BATCH, TAIL)], sem1).wait()

    @pl.when(wid == 1)
    def _():
        pltpu.sync_copy(t_h, tvec)
        d1 = pltpu.async_copy(bt_h.at[pl.ds(BATCH, TAIL)],
                              stage.at[pl.ds(BATCH, TAIL)], sem1)
        tv = tvec[...]
        for i in range(BATCH // 16):
            stage[pl.ds(i * 16, 16)] = tv
        pltpu.async_copy(stage.at[pl.ds(0, BATCH)],
                         obt_h.at[pl.ds(0, BATCH)], sem0).wait()
        d1.wait()
        pltpu.async_copy(stage.at[pl.ds(BATCH, TAIL)],
                         obt_h.at[pl.ds(BATCH, TAIL)], sem1).wait()


@jax.jit
def _er_update(x, y, t16, bx, by, bt):
    xf = x.reshape(R1)
    bxf = bx.reshape(TOT)

    obx = pl.pallas_call(
        _bx_body,
        in_specs=[pl.BlockSpec(memory_space=pltpu.MemorySpace.HBM),
                  pl.BlockSpec(memory_space=pltpu.MemorySpace.HBM)],
        out_specs=pl.BlockSpec(memory_space=pltpu.MemorySpace.HBM),
        out_shape=jax.ShapeDtypeStruct((TOT,), jnp.float32),
        scratch_shapes=(
            [pltpu.VMEM((CHUNK,), jnp.float32) for _ in range(NB)]
            + [pltpu.SemaphoreType.DMA((NB,)),
               pltpu.SemaphoreType.DMA((NB,))]),
        compiler_params=pltpu.CompilerParams(
            vmem_limit_bytes=100 * 1024 * 1024),
    )(xf, bxf)

    mesh = plsc.VectorSubcoreMesh(core_axis_name="c", subcore_axis_name="s")
    oby, obt = pl.kernel(
        _sc_body,
        out_type=(
            jax.ShapeDtypeStruct((BUFFER_SIZE,), jnp.int32),
            jax.ShapeDtypeStruct((BUFFER_SIZE,), jnp.int32),
        ),
        mesh=mesh,
        scratch_types=[
            pltpu.VMEM((BUFFER_SIZE,), jnp.int32),
            pltpu.VMEM((16,), jnp.int32),
            pltpu.SemaphoreType.DMA,
            pltpu.SemaphoreType.DMA,
        ],
    )(y, t16, by, bt)
    return obx.reshape(bx.shape), oby, obt


def kernel(x, y, task_id, bx, by, bt):
    t16 = jnp.full((16,), task_id, dtype=jnp.int32)
    return _er_update(x, y, t16, bx, by, bt)
